# Initial kernel scaffold; baseline (speedup 1.0000x reference)
import jax
import jax.numpy as jnp
from jax.experimental import pallas as pl


def kernel(x, edge_index, W1, a_src1, a_dst1, b1, W2, a_src2, a_dst2, b2, Wih_f, Whh_f, bih_f, bhh_f, Wih_r, Whh_r, bih_r, bhh_r, Wp):
    raise NotImplementedError("write your pallas kernel here")


# speedup vs baseline: 9.8064x; 9.8064x over previous
import functools

import jax
import jax.numpy as jnp
from jax import lax
from jax.experimental import pallas as pl
from jax.experimental.pallas import tpu as pltpu
from jax.experimental.pallas import tpu_sc as plsc

N = 10000
E = 320000
DIN = 128
D = 256
DH = 128
DP = 144
BM = 400
GRID = N // BM
NS = 16
L = 16
EPT = E // NS
CH = 80
NCHUNK = EPT // CH
RPT = N // NS
F32 = jnp.float32


def _leaky(v):
    return jnp.where(v >= 0, v, 0.2 * v)


def _prep_body(x_ref, w_ref, asv_ref, adv_ref, ha_ref, hb_ref, asad_ref):
    h = jnp.dot(x_ref[...], w_ref[...], preferred_element_type=F32)
    a_s = jnp.dot(h, asv_ref[...], preferred_element_type=F32)
    a_d = jnp.dot(h, adv_ref[...], preferred_element_type=F32)
    tail_i = lax.broadcasted_iota(jnp.int32, (BM, DP - DH), 1)
    tail = jnp.where(tail_i == 0, 1.0, 0.0).astype(F32)
    ha_ref[:, :DH] = h[:, :DH]
    ha_ref[:, DH:] = tail
    hb_ref[:, :DH] = h[:, DH:]
    hb_ref[:, DH:] = tail
    t8 = lax.broadcasted_iota(jnp.int32, (BM, 8), 1)
    asad_ref[...] = jnp.where(t8 == 0, a_s, jnp.where(t8 == 1, a_d, 0.0)).astype(F32)


def _prep(x, w, asv, adv):
    return pl.pallas_call(
        _prep_body,
        grid=(GRID,),
        in_specs=[
            pl.BlockSpec((BM, DIN), lambda i: (i, 0)),
            pl.BlockSpec((DIN, D), lambda i: (0, 0)),
            pl.BlockSpec((D, 1), lambda i: (0, 0)),
            pl.BlockSpec((D, 1), lambda i: (0, 0)),
        ],
        out_specs=[
            pl.BlockSpec((BM, DP), lambda i: (i, 0)),
            pl.BlockSpec((BM, DP), lambda i: (i, 0)),
            pl.BlockSpec((BM, 8), lambda i: (i, 0)),
        ],
        out_shape=[
            jax.ShapeDtypeStruct((N, DP), F32),
            jax.ShapeDtypeStruct((N, DP), F32),
            jax.ShapeDtypeStruct((N, 8), F32),
        ],
    )(x, w, asv, adv)


def _norm_body(sa_ref, sb_ref, ha_ref, hb_ref, asad_ref, b_ref, w2_ref,
               asv_ref, adv_ref, x1_ref, ha2_ref, hb2_ref, asad2_ref):
    ws = jnp.exp(_leaky(asad_ref[:, 0:1] + asad_ref[:, 1:2]))
    den = sa_ref[:, DH:DH + 1] + ws + 1e-16
    xa = (sa_ref[:, :DH] + ws * ha_ref[:, :DH]) / den + b_ref[:, :DH]
    xb = (sb_ref[:, :DH] + ws * hb_ref[:, :DH]) / den + b_ref[:, DH:]
    x1 = jnp.maximum(jnp.concatenate([xa, xb], axis=1), 0.0)
    x1_ref[...] = x1
    h2 = jnp.dot(x1, w2_ref[...], preferred_element_type=F32)
    a_s = jnp.dot(h2, asv_ref[...], preferred_element_type=F32)
    a_d = jnp.dot(h2, adv_ref[...], preferred_element_type=F32)
    tail_i = lax.broadcasted_iota(jnp.int32, (BM, DP - DH), 1)
    tail = jnp.where(tail_i == 0, 1.0, 0.0).astype(F32)
    ha2_ref[:, :DH] = h2[:, :DH]
    ha2_ref[:, DH:] = tail
    hb2_ref[:, :DH] = h2[:, DH:]
    hb2_ref[:, DH:] = tail
    t8 = lax.broadcasted_iota(jnp.int32, (BM, 8), 1)
    asad2_ref[...] = jnp.where(t8 == 0, a_s, jnp.where(t8 == 1, a_d, 0.0)).astype(F32)


def _norm(sa, sb, ha, hb, asad, b, w2, asv, adv):
    return pl.pallas_call(
        _norm_body,
        grid=(GRID,),
        in_specs=[
            pl.BlockSpec((BM, DP), lambda i: (i, 0)),
            pl.BlockSpec((BM, DP), lambda i: (i, 0)),
            pl.BlockSpec((BM, DP), lambda i: (i, 0)),
            pl.BlockSpec((BM, DP), lambda i: (i, 0)),
            pl.BlockSpec((BM, 8), lambda i: (i, 0)),
            pl.BlockSpec((1, D), lambda i: (0, 0)),
            pl.BlockSpec((D, D), lambda i: (0, 0)),
            pl.BlockSpec((D, 1), lambda i: (0, 0)),
            pl.BlockSpec((D, 1), lambda i: (0, 0)),
        ],
        out_specs=[
            pl.BlockSpec((BM, D), lambda i: (i, 0)),
            pl.BlockSpec((BM, DP), lambda i: (i, 0)),
            pl.BlockSpec((BM, DP), lambda i: (i, 0)),
            pl.BlockSpec((BM, 8), lambda i: (i, 0)),
        ],
        out_shape=[
            jax.ShapeDtypeStruct((N, D), F32),
            jax.ShapeDtypeStruct((N, DP), F32),
            jax.ShapeDtypeStruct((N, DP), F32),
            jax.ShapeDtypeStruct((N, 8), F32),
        ],
    )(sa, sb, ha, hb, asad, b, w2, asv, adv)


def _final_body(sa_ref, sb_ref, ha_ref, hb_ref, asad_ref, b_ref, x1_ref,
                wihf_ref, whhf_ref, wihr_ref, whhr_ref,
                bihf_ref, bhhf_ref, bihr_ref, bhhr_ref,
                wpa_ref, wpb_ref, out_ref):
    ws = jnp.exp(_leaky(asad_ref[:, 0:1] + asad_ref[:, 1:2]))
    den = sa_ref[:, DH:DH + 1] + ws + 1e-16
    xa = (sa_ref[:, :DH] + ws * ha_ref[:, :DH]) / den + b_ref[:, :DH]
    xb = (sb_ref[:, :DH] + ws * hb_ref[:, :DH]) / den + b_ref[:, DH:]
    x2 = jnp.maximum(jnp.concatenate([xa, xb], axis=1), 0.0)
    x1 = x1_ref[...]

    bf = bihf_ref[...] + bhhf_ref[...]
    br = bihr_ref[...] + bhhr_ref[...]

    def cell(xt, hprev, cprev, wih_ref, whh_ref, bb):
        g = jnp.dot(xt, wih_ref[...], preferred_element_type=F32) + bb
        if hprev is not None:
            g = g + jnp.dot(hprev, whh_ref[...], preferred_element_type=F32)
        i = jax.nn.sigmoid(g[:, :D])
        f = jax.nn.sigmoid(g[:, D:2 * D])
        gg = jnp.tanh(g[:, 2 * D:3 * D])
        o = jax.nn.sigmoid(g[:, 3 * D:])
        c = i * gg
        if cprev is not None:
            c = c + f * cprev
        return o * jnp.tanh(c), c

    hf1, cf1 = cell(x1, None, None, wihf_ref, whhf_ref, bf)
    hf2, _ = cell(x2, hf1, cf1, wihf_ref, whhf_ref, bf)
    hr2, cr2 = cell(x2, None, None, wihr_ref, whhr_ref, br)
    hr1, _ = cell(x1, hr2, cr2, wihr_ref, whhr_ref, br)

    s1 = (jnp.dot(hf1, wpa_ref[...], preferred_element_type=F32)
          + jnp.dot(hr1, wpb_ref[...], preferred_element_type=F32))
    s2 = (jnp.dot(hf2, wpa_ref[...], preferred_element_type=F32)
          + jnp.dot(hr2, wpb_ref[...], preferred_element_type=F32))
    a1 = jax.nn.sigmoid(s1 - s2)
    a2 = jax.nn.sigmoid(s2 - s1)
    out_ref[...] = a1 * x1 + a2 * x2


def _final(sa, sb, ha, hb, asad, b, x1, wihf, whhf, wihr, whhr,
           bihf, bhhf, bihr, bhhr, wpa, wpb):
    c0 = lambda i: (0, 0)
    return pl.pallas_call(
        _final_body,
        grid=(GRID,),
        in_specs=[
            pl.BlockSpec((BM, DP), lambda i: (i, 0)),
            pl.BlockSpec((BM, DP), lambda i: (i, 0)),
            pl.BlockSpec((BM, DP), lambda i: (i, 0)),
            pl.BlockSpec((BM, DP), lambda i: (i, 0)),
            pl.BlockSpec((BM, 8), lambda i: (i, 0)),
            pl.BlockSpec((1, D), c0),
            pl.BlockSpec((BM, D), lambda i: (i, 0)),
            pl.BlockSpec((D, 4 * D), c0),
            pl.BlockSpec((D, 4 * D), c0),
            pl.BlockSpec((D, 4 * D), c0),
            pl.BlockSpec((D, 4 * D), c0),
            pl.BlockSpec((1, 4 * D), c0),
            pl.BlockSpec((1, 4 * D), c0),
            pl.BlockSpec((1, 4 * D), c0),
            pl.BlockSpec((1, 4 * D), c0),
            pl.BlockSpec((D, 1), c0),
            pl.BlockSpec((D, 1), c0),
        ],
        out_specs=pl.BlockSpec((BM, D), lambda i: (i, 0)),
        out_shape=jax.ShapeDtypeStruct((N, D), F32),
    )(sa, sb, ha, hb, asad, b, x1, wihf, whhf, wihr, whhr,
      bihf, bhhf, bihr, bhhr, wpa, wpb)


def _build_edge_kernel():
    mesh = plsc.VectorSubcoreMesh(core_axis_name="c", subcore_axis_name="s")

    @functools.partial(
        pl.kernel,
        out_type=(jax.ShapeDtypeStruct((N, DP), F32),
                  jax.ShapeDtypeStruct((N, DP), F32)),
        mesh=mesh,
        scratch_types=[
            pltpu.VMEM((CH, 8), F32),
            pltpu.VMEM((CH, 8), F32),
            pltpu.VMEM((CH,), jnp.int32),
            pltpu.VMEM((CH,), jnp.int32),
            pltpu.VMEM((CH,), jnp.int32),
            pltpu.VMEM((CH,), F32),
            pltpu.VMEM((CH, DP), F32),
            pltpu.VMEM_SHARED((N, DP), F32),
            pltpu.SemaphoreType.DMA,
        ],
        compiler_params=pltpu.CompilerParams(use_tc_tiling_on_sc=False,
                                             needs_layout_passes=False),
    )
    def edge_kernel(hs_hbm, src_hbm, dst_hbm, asad_hbm, sa_out, sb_out,
                    asv_v, adv_v, src_v, srcg_v, dst_v, w_v, rows_v, s_sh, sem):
        c = lax.axis_index("c")
        s = lax.axis_index("s")

        def zero_body(r, _):
            for g in range(DP // L):
                rows_v[r, pl.ds(g * L, L)] = jnp.zeros((L,), F32)
            return 0
        lax.fori_loop(0, CH, zero_body, 0)
        row0 = pl.multiple_of(s * RPT, RPT)
        for j in range(RPT // CH):
            pltpu.sync_copy(rows_v, s_sh.at[pl.ds(row0 + j * CH, CH)])
        rem = RPT - (RPT // CH) * CH
        if rem:
            pltpu.sync_copy(rows_v.at[pl.ds(0, rem)],
                            s_sh.at[pl.ds(row0 + (RPT // CH) * CH, rem)])
        plsc.subcore_barrier()

        ebase = pl.multiple_of(s * EPT, EPT)
        z16 = jnp.zeros((L,), jnp.int32)
        o16 = jnp.ones((L,), jnp.int32)
        i16 = lax.iota(jnp.int32, L)

        def chunk_body(jj, _):
            off = pl.multiple_of(ebase + jj * CH, CH)
            pltpu.sync_copy(src_hbm.at[pl.ds(off, CH)], src_v)
            pltpu.sync_copy(dst_hbm.at[pl.ds(off, CH)], dst_v)
            pltpu.async_copy(asad_hbm.at[src_v], asv_v, sem).wait()
            pltpu.async_copy(asad_hbm.at[dst_v], adv_v, sem).wait()
            for g in range(CH // L):
                srcg_v[pl.ds(g * L, L)] = src_v[pl.ds(g * L, L)] + c * N
            pltpu.async_copy(hs_hbm.at[srcg_v], rows_v, sem).wait()

            for g in range(CH // L):
                ridx = i16 + g * L
                av = plsc.load_gather(asv_v, [ridx, z16])
                dv = plsc.load_gather(adv_v, [ridx, o16])
                e = _leaky(av + dv)
                w_v[pl.ds(g * L, L)] = jnp.exp(e)

            def scale_body(k, _):
                wsp = plsc.load_gather(w_v, [jnp.full((L,), k, jnp.int32)])
                for g in range(DP // L):
                    rows_v[k, pl.ds(g * L, L)] = rows_v[k, pl.ds(g * L, L)] * wsp
                return 0
            lax.fori_loop(0, CH, scale_body, 0)

            pltpu.sync_copy(rows_v, s_sh.at[dst_v], add=True)
            return 0
        lax.fori_loop(0, NCHUNK, chunk_body, 0)
        plsc.subcore_barrier()

        @pl.when(c == 0)
        def _():
            pltpu.sync_copy(s_sh.at[pl.ds(row0, RPT)], sa_out.at[pl.ds(row0, RPT)])

        @pl.when(c == 1)
        def _():
            pltpu.sync_copy(s_sh.at[pl.ds(row0, RPT)], sb_out.at[pl.ds(row0, RPT)])

    return edge_kernel


_edge_cache = []


def _edge_pass(ha, hb, src, dst, asad):
    if not _edge_cache:
        _edge_cache.append(_build_edge_kernel())
    hs = jnp.concatenate([ha, hb], axis=0)
    return _edge_cache[0](hs, src, dst, asad)


def kernel(x, edge_index, W1, a_src1, a_dst1, b1, W2, a_src2, a_dst2, b2,
           Wih_f, Whh_f, bih_f, bhh_f, Wih_r, Whh_r, bih_r, bhh_r, Wp):
    src = edge_index[0]
    dst = edge_index[1]
    ha1, hb1, asad1 = _prep(x, W1, a_src1.reshape(-1, 1), a_dst1.reshape(-1, 1))
    sa1, sb1 = _edge_pass(ha1, hb1, src, dst, asad1)
    x1, ha2, hb2, asad2 = _norm(sa1, sb1, ha1, hb1, asad1, b1.reshape(1, -1),
                                W2, a_src2.reshape(-1, 1), a_dst2.reshape(-1, 1))
    sa2, sb2 = _edge_pass(ha2, hb2, src, dst, asad2)
    out = _final(sa2, sb2, ha2, hb2, asad2, b2.reshape(1, -1), x1,
                 Wih_f.T, Whh_f.T, Wih_r.T, Whh_r.T,
                 bih_f.reshape(1, -1), bhh_f.reshape(1, -1),
                 bih_r.reshape(1, -1), bhh_r.reshape(1, -1),
                 Wp[:, :D].reshape(-1, 1), Wp[:, D:].reshape(-1, 1))
    return out


# speedup vs baseline: 21.3330x; 2.1754x over previous
import functools

import jax
import jax.numpy as jnp
from jax import lax
from jax.experimental import pallas as pl
from jax.experimental.pallas import tpu as pltpu
from jax.experimental.pallas import tpu_sc as plsc

N = 10000
E = 320000
DIN = 128
D = 256
DH = 128
DP = 144
BM = 400
GRID = N // BM
NS = 16
L = 16
EPT = E // NS
CH = 80
NCHUNK = EPT // CH
RPT = N // NS
F32 = jnp.float32


def _leaky(v):
    return jnp.where(v >= 0, v, 0.2 * v)


def _prep_body(x_ref, w_ref, asv_ref, adv_ref, ha_ref, hb_ref, asad_ref):
    h = jnp.dot(x_ref[...], w_ref[...], preferred_element_type=F32)
    a_s = jnp.dot(h, asv_ref[...], preferred_element_type=F32)
    a_d = jnp.dot(h, adv_ref[...], preferred_element_type=F32)
    tail_i = lax.broadcasted_iota(jnp.int32, (BM, DP - DH), 1)
    tail = jnp.where(tail_i == 0, 1.0, 0.0).astype(F32)
    ha_ref[:, :DH] = h[:, :DH]
    ha_ref[:, DH:] = tail
    hb_ref[:, :DH] = h[:, DH:]
    hb_ref[:, DH:] = tail
    t8 = lax.broadcasted_iota(jnp.int32, (BM, 8), 1)
    asad_ref[...] = jnp.where(t8 == 0, a_s, jnp.where(t8 == 1, a_d, 0.0)).astype(F32)


def _prep(x, w, asv, adv):
    return pl.pallas_call(
        _prep_body,
        grid=(GRID,),
        in_specs=[
            pl.BlockSpec((BM, DIN), lambda i: (i, 0)),
            pl.BlockSpec((DIN, D), lambda i: (0, 0)),
            pl.BlockSpec((D, 1), lambda i: (0, 0)),
            pl.BlockSpec((D, 1), lambda i: (0, 0)),
        ],
        out_specs=[
            pl.BlockSpec((BM, DP), lambda i: (i, 0)),
            pl.BlockSpec((BM, DP), lambda i: (i, 0)),
            pl.BlockSpec((BM, 8), lambda i: (i, 0)),
        ],
        out_shape=[
            jax.ShapeDtypeStruct((N, DP), F32),
            jax.ShapeDtypeStruct((N, DP), F32),
            jax.ShapeDtypeStruct((N, 8), F32),
        ],
    )(x, w, asv, adv)


def _norm_body(sa_ref, sb_ref, ha_ref, hb_ref, asad_ref, b_ref, w2_ref,
               asv_ref, adv_ref, x1_ref, ha2_ref, hb2_ref, asad2_ref):
    ws = jnp.exp(_leaky(asad_ref[:, 0:1] + asad_ref[:, 1:2]))
    den = sa_ref[:, DH:DH + 1] + ws + 1e-16
    xa = (sa_ref[:, :DH] + ws * ha_ref[:, :DH]) / den + b_ref[:, :DH]
    xb = (sb_ref[:, :DH] + ws * hb_ref[:, :DH]) / den + b_ref[:, DH:]
    x1 = jnp.maximum(jnp.concatenate([xa, xb], axis=1), 0.0)
    x1_ref[...] = x1
    h2 = jnp.dot(x1, w2_ref[...], preferred_element_type=F32)
    a_s = jnp.dot(h2, asv_ref[...], preferred_element_type=F32)
    a_d = jnp.dot(h2, adv_ref[...], preferred_element_type=F32)
    tail_i = lax.broadcasted_iota(jnp.int32, (BM, DP - DH), 1)
    tail = jnp.where(tail_i == 0, 1.0, 0.0).astype(F32)
    ha2_ref[:, :DH] = h2[:, :DH]
    ha2_ref[:, DH:] = tail
    hb2_ref[:, :DH] = h2[:, DH:]
    hb2_ref[:, DH:] = tail
    t8 = lax.broadcasted_iota(jnp.int32, (BM, 8), 1)
    asad2_ref[...] = jnp.where(t8 == 0, a_s, jnp.where(t8 == 1, a_d, 0.0)).astype(F32)


def _norm(sa, sb, ha, hb, asad, b, w2, asv, adv):
    return pl.pallas_call(
        _norm_body,
        grid=(GRID,),
        in_specs=[
            pl.BlockSpec((BM, DP), lambda i: (i, 0)),
            pl.BlockSpec((BM, DP), lambda i: (i, 0)),
            pl.BlockSpec((BM, DP), lambda i: (i, 0)),
            pl.BlockSpec((BM, DP), lambda i: (i, 0)),
            pl.BlockSpec((BM, 8), lambda i: (i, 0)),
            pl.BlockSpec((1, D), lambda i: (0, 0)),
            pl.BlockSpec((D, D), lambda i: (0, 0)),
            pl.BlockSpec((D, 1), lambda i: (0, 0)),
            pl.BlockSpec((D, 1), lambda i: (0, 0)),
        ],
        out_specs=[
            pl.BlockSpec((BM, D), lambda i: (i, 0)),
            pl.BlockSpec((BM, DP), lambda i: (i, 0)),
            pl.BlockSpec((BM, DP), lambda i: (i, 0)),
            pl.BlockSpec((BM, 8), lambda i: (i, 0)),
        ],
        out_shape=[
            jax.ShapeDtypeStruct((N, D), F32),
            jax.ShapeDtypeStruct((N, DP), F32),
            jax.ShapeDtypeStruct((N, DP), F32),
            jax.ShapeDtypeStruct((N, 8), F32),
        ],
    )(sa, sb, ha, hb, asad, b, w2, asv, adv)


def _final_body(sa_ref, sb_ref, ha_ref, hb_ref, asad_ref, b_ref, x1_ref,
                wihf_ref, whhf_ref, wihr_ref, whhr_ref,
                bihf_ref, bhhf_ref, bihr_ref, bhhr_ref,
                wpa_ref, wpb_ref, out_ref):
    ws = jnp.exp(_leaky(asad_ref[:, 0:1] + asad_ref[:, 1:2]))
    den = sa_ref[:, DH:DH + 1] + ws + 1e-16
    xa = (sa_ref[:, :DH] + ws * ha_ref[:, :DH]) / den + b_ref[:, :DH]
    xb = (sb_ref[:, :DH] + ws * hb_ref[:, :DH]) / den + b_ref[:, DH:]
    x2 = jnp.maximum(jnp.concatenate([xa, xb], axis=1), 0.0)
    x1 = x1_ref[...]

    bf = bihf_ref[...] + bhhf_ref[...]
    br = bihr_ref[...] + bhhr_ref[...]

    def cell(xt, hprev, cprev, wih_ref, whh_ref, bb):
        g = jnp.dot(xt, wih_ref[...], preferred_element_type=F32) + bb
        if hprev is not None:
            g = g + jnp.dot(hprev, whh_ref[...], preferred_element_type=F32)
        i = jax.nn.sigmoid(g[:, :D])
        f = jax.nn.sigmoid(g[:, D:2 * D])
        gg = jnp.tanh(g[:, 2 * D:3 * D])
        o = jax.nn.sigmoid(g[:, 3 * D:])
        c = i * gg
        if cprev is not None:
            c = c + f * cprev
        return o * jnp.tanh(c), c

    hf1, cf1 = cell(x1, None, None, wihf_ref, whhf_ref, bf)
    hf2, _ = cell(x2, hf1, cf1, wihf_ref, whhf_ref, bf)
    hr2, cr2 = cell(x2, None, None, wihr_ref, whhr_ref, br)
    hr1, _ = cell(x1, hr2, cr2, wihr_ref, whhr_ref, br)

    s1 = (jnp.dot(hf1, wpa_ref[...], preferred_element_type=F32)
          + jnp.dot(hr1, wpb_ref[...], preferred_element_type=F32))
    s2 = (jnp.dot(hf2, wpa_ref[...], preferred_element_type=F32)
          + jnp.dot(hr2, wpb_ref[...], preferred_element_type=F32))
    a1 = jax.nn.sigmoid(s1 - s2)
    a2 = jax.nn.sigmoid(s2 - s1)
    out_ref[...] = a1 * x1 + a2 * x2


def _final(sa, sb, ha, hb, asad, b, x1, wihf, whhf, wihr, whhr,
           bihf, bhhf, bihr, bhhr, wpa, wpb):
    c0 = lambda i: (0, 0)
    return pl.pallas_call(
        _final_body,
        grid=(GRID,),
        in_specs=[
            pl.BlockSpec((BM, DP), lambda i: (i, 0)),
            pl.BlockSpec((BM, DP), lambda i: (i, 0)),
            pl.BlockSpec((BM, DP), lambda i: (i, 0)),
            pl.BlockSpec((BM, DP), lambda i: (i, 0)),
            pl.BlockSpec((BM, 8), lambda i: (i, 0)),
            pl.BlockSpec((1, D), c0),
            pl.BlockSpec((BM, D), lambda i: (i, 0)),
            pl.BlockSpec((D, 4 * D), c0),
            pl.BlockSpec((D, 4 * D), c0),
            pl.BlockSpec((D, 4 * D), c0),
            pl.BlockSpec((D, 4 * D), c0),
            pl.BlockSpec((1, 4 * D), c0),
            pl.BlockSpec((1, 4 * D), c0),
            pl.BlockSpec((1, 4 * D), c0),
            pl.BlockSpec((1, 4 * D), c0),
            pl.BlockSpec((D, 1), c0),
            pl.BlockSpec((D, 1), c0),
        ],
        out_specs=pl.BlockSpec((BM, D), lambda i: (i, 0)),
        out_shape=jax.ShapeDtypeStruct((N, D), F32),
    )(sa, sb, ha, hb, asad, b, x1, wihf, whhf, wihr, whhr,
      bihf, bhhf, bihr, bhhr, wpa, wpb)


def _build_edge_kernel():
    mesh = plsc.VectorSubcoreMesh(core_axis_name="c", subcore_axis_name="s")

    @functools.partial(
        pl.kernel,
        out_type=(jax.ShapeDtypeStruct((N, DP), F32),
                  jax.ShapeDtypeStruct((N, DP), F32)),
        mesh=mesh,
        scratch_types=[
            [pltpu.VMEM((CH, 8), F32)] * 2,
            [pltpu.VMEM((CH, 8), F32)] * 2,
            [pltpu.VMEM((CH,), jnp.int32)] * 2,
            [pltpu.VMEM((CH,), jnp.int32)] * 2,
            [pltpu.VMEM((CH,), jnp.int32)] * 2,
            [pltpu.VMEM((CH,), jnp.int32)] * 2,
            pltpu.VMEM((CH,), F32),
            [pltpu.VMEM((CH, DP), F32)] * 2,
            pltpu.VMEM_SHARED((N, DP), F32),
            [pltpu.SemaphoreType.DMA] * 2,
            [pltpu.SemaphoreType.DMA] * 2,
        ],
        compiler_params=pltpu.CompilerParams(use_tc_tiling_on_sc=False,
                                             needs_layout_passes=False),
    )
    def edge_kernel(hs_hbm, src_hbm, dst_hbm, asad_hbm, sa_out, sb_out,
                    asv_v, adv_v, src_v, srcg_v, dst_v, dsts_v, w_v, rows_v,
                    s_sh, sem_g, sem_s):
        c = lax.axis_index("c")
        s = lax.axis_index("s")

        def zero_body(r, _):
            for g in range(DP // L):
                rows_v[0][r, pl.ds(g * L, L)] = jnp.zeros((L,), F32)
            return 0
        lax.fori_loop(0, CH, zero_body, 0)
        row0 = pl.multiple_of(s * RPT, RPT)
        for j in range(RPT // CH):
            pltpu.sync_copy(rows_v[0], s_sh.at[pl.ds(row0 + j * CH, CH)])
        rem = RPT - (RPT // CH) * CH
        if rem:
            pltpu.sync_copy(rows_v[0].at[pl.ds(0, rem)],
                            s_sh.at[pl.ds(row0 + (RPT // CH) * CH, rem)])
        plsc.subcore_barrier()

        ebase = pl.multiple_of(s * EPT, EPT)
        z16 = jnp.zeros((L,), jnp.int32)
        o16 = jnp.ones((L,), jnp.int32)
        i16 = lax.iota(jnp.int32, L)

        def load_idx(j, b):
            jc = jnp.minimum(j, NCHUNK - 1)
            off = pl.multiple_of(ebase + jc * CH, CH)
            pltpu.sync_copy(src_hbm.at[pl.ds(off, CH)], src_v[b])
            pltpu.sync_copy(dst_hbm.at[pl.ds(off, CH)], dst_v[b])
            for g in range(CH // L):
                srcg_v[b][pl.ds(g * L, L)] = src_v[b][pl.ds(g * L, L)] + c * N

        def fire_gathers(b):
            pltpu.async_copy(hs_hbm.at[srcg_v[b]], rows_v[b], sem_g[b])
            pltpu.async_copy(asad_hbm.at[src_v[b]], asv_v[b], sem_g[b])
            pltpu.async_copy(asad_hbm.at[dst_v[b]], adv_v[b], sem_g[b])

        def wait_gathers(b):
            pltpu.make_async_copy(hs_hbm.at[srcg_v[b]], rows_v[b], sem_g[b]).wait()
            pltpu.make_async_copy(asad_hbm.at[src_v[b]], asv_v[b], sem_g[b]).wait()
            pltpu.make_async_copy(asad_hbm.at[dst_v[b]], adv_v[b], sem_g[b]).wait()

        def wait_scatter(b):
            pltpu.make_async_copy(rows_v[b], s_sh.at[dsts_v[b]], sem_s[b]).wait()

        load_idx(jnp.int32(0), 0)
        fire_gathers(0)

        def outer_body(jo, _):
            for b in range(2):
                nb = 1 - b
                j = jo * 2 + b
                load_idx(j + 1, nb)
                wait_gathers(b)
                if b == 0:
                    @pl.when(jo > 0)
                    def _():
                        wait_scatter(nb)
                else:
                    wait_scatter(nb)
                fire_gathers(nb)
                for g in range(CH // L):
                    ridx = i16 + g * L
                    av = plsc.load_gather(asv_v[b], [ridx, z16])
                    dv = plsc.load_gather(adv_v[b], [ridx, o16])
                    e = _leaky(av + dv)
                    w_v[pl.ds(g * L, L)] = jnp.exp(e)

                def scale_body(k, _):
                    wsp = plsc.load_gather(w_v, [jnp.full((L,), k, jnp.int32)])
                    for g in range(DP // L):
                        rows_v[b][k, pl.ds(g * L, L)] = (
                            rows_v[b][k, pl.ds(g * L, L)] * wsp)
                    return 0
                lax.fori_loop(0, CH, scale_body, 0)
                for g in range(CH // L):
                    dsts_v[b][pl.ds(g * L, L)] = dst_v[b][pl.ds(g * L, L)]
                pltpu.async_copy(rows_v[b], s_sh.at[dsts_v[b]], sem_s[b],
                                 add=True)
            return 0
        lax.fori_loop(0, NCHUNK // 2, outer_body, 0)
        wait_gathers(0)
        wait_scatter(1)
        plsc.subcore_barrier()

        @pl.when(c == 0)
        def _():
            pltpu.sync_copy(s_sh.at[pl.ds(row0, RPT)], sa_out.at[pl.ds(row0, RPT)])

        @pl.when(c == 1)
        def _():
            pltpu.sync_copy(s_sh.at[pl.ds(row0, RPT)], sb_out.at[pl.ds(row0, RPT)])

    return edge_kernel


_edge_cache = []


def _edge_pass(ha, hb, src, dst, asad):
    if not _edge_cache:
        _edge_cache.append(_build_edge_kernel())
    hs = jnp.concatenate([ha, hb], axis=0)
    return _edge_cache[0](hs, src, dst, asad)


def kernel(x, edge_index, W1, a_src1, a_dst1, b1, W2, a_src2, a_dst2, b2,
           Wih_f, Whh_f, bih_f, bhh_f, Wih_r, Whh_r, bih_r, bhh_r, Wp):
    src = edge_index[0]
    dst = edge_index[1]
    ha1, hb1, asad1 = _prep(x, W1, a_src1.reshape(-1, 1), a_dst1.reshape(-1, 1))
    sa1, sb1 = _edge_pass(ha1, hb1, src, dst, asad1)
    x1, ha2, hb2, asad2 = _norm(sa1, sb1, ha1, hb1, asad1, b1.reshape(1, -1),
                                W2, a_src2.reshape(-1, 1), a_dst2.reshape(-1, 1))
    sa2, sb2 = _edge_pass(ha2, hb2, src, dst, asad2)
    out = _final(sa2, sb2, ha2, hb2, asad2, b2.reshape(1, -1), x1,
                 Wih_f.T, Whh_f.T, Wih_r.T, Whh_r.T,
                 bih_f.reshape(1, -1), bhh_f.reshape(1, -1),
                 bih_r.reshape(1, -1), bhh_r.reshape(1, -1),
                 Wp[:, :D].reshape(-1, 1), Wp[:, D:].reshape(-1, 1))
    return out
